# baseline (speedup 1.0000x reference)
import functools

import jax
import jax.numpy as jnp
from jax import lax
from jax.experimental import pallas as pl
from jax.experimental.pallas import tpu as pltpu
from jax.experimental.pallas import tpu_sc as plsc

DIM = 1024
NC = 2
NS = 16
NW = NC * NS
CHUNK = 32
NBUF = 3
LOOKAHEAD = 1


def _make_gather(B: int, D: int):
  b_per_w = B // NW
  n_chunks = b_per_w // CHUNK
  mesh = plsc.VectorSubcoreMesh(core_axis_name="c", subcore_axis_name="s")

  @functools.partial(
      pl.kernel,
      mesh=mesh,
      out_type=jax.ShapeDtypeStruct((B, D), jnp.float32),
      scratch_types=(
          [pltpu.VMEM((b_per_w,), jnp.int32)]
          + [pltpu.VMEM((CHUNK, D), jnp.float32)] * NBUF
          + [pltpu.SemaphoreType.DMA] * (2 * NBUF)
      ),
  )
  def k(table_hbm, idx_hbm, out_hbm, idx_v, *bufsems):
    bufs = bufsems[:NBUF]
    gsem = bufsems[NBUF:2 * NBUF]
    ssem = bufsems[2 * NBUF:]
    wid = lax.axis_index("s") * NC + lax.axis_index("c")
    base = wid * b_per_w
    pltpu.sync_copy(idx_hbm.at[pl.ds(base, b_per_w)], idx_v)

    def start_gather(c, b):
      pltpu.async_copy(
          table_hbm.at[idx_v.at[pl.ds(c * CHUNK, CHUNK)]], bufs[b], gsem[b])

    def wait_gather(b):
      pltpu.make_async_copy(
          table_hbm.at[idx_v.at[pl.ds(0, CHUNK)]], bufs[b], gsem[b]).wait()

    def start_scatter(c, b):
      pltpu.async_copy(
          bufs[b], out_hbm.at[pl.ds(base + c * CHUNK, CHUNK)], ssem[b])

    def wait_scatter(b):
      pltpu.make_async_copy(
          bufs[b], out_hbm.at[pl.ds(base, CHUNK)], ssem[b]).wait()

    for c in range(LOOKAHEAD):
      start_gather(c, c % NBUF)
    for c in range(n_chunks):
      cg = c + LOOKAHEAD
      if cg < n_chunks:
        bg = cg % NBUF
        if cg >= NBUF:
          wait_scatter(bg)
        start_gather(cg, bg)
      b = c % NBUF
      wait_gather(b)
      start_scatter(c, b)

    for b in range(NBUF):
      wait_scatter(b)

  return k


def kernel(tok_idx, embeddings):
  bsz, seqlen = tok_idx.shape
  flat_idx = tok_idx.reshape(bsz * seqlen)
  out = _make_gather(bsz * seqlen, DIM)(embeddings, flat_idx)
  return out.reshape(bsz, seqlen, DIM)


# speedup vs baseline: 1.0126x; 1.0126x over previous
import functools

import jax
import jax.numpy as jnp
from jax import lax
from jax.experimental import pallas as pl
from jax.experimental.pallas import tpu as pltpu
from jax.experimental.pallas import tpu_sc as plsc

DIM = 1024
NC = 2
NS = 16
NW = NC * NS
CHUNK = 56
NBUF = 2
LOOKAHEAD = 1


def _make_gather(B: int, D: int):
  b_per_w = B // NW
  sizes = [CHUNK] * (b_per_w // CHUNK) + ([b_per_w % CHUNK] if b_per_w % CHUNK else [])
  offs = [sum(sizes[:i]) for i in range(len(sizes))]
  n_chunks = len(sizes)
  mesh = plsc.VectorSubcoreMesh(core_axis_name="c", subcore_axis_name="s")

  @functools.partial(
      pl.kernel,
      mesh=mesh,
      out_type=jax.ShapeDtypeStruct((B, D), jnp.float32),
      scratch_types=(
          [pltpu.VMEM((b_per_w,), jnp.int32)]
          + [pltpu.VMEM((CHUNK, D), jnp.float32)] * NBUF
          + [pltpu.SemaphoreType.DMA] * (2 * NBUF)
      ),
  )
  def k(table_hbm, idx_hbm, out_hbm, idx_v, *bufsems):
    bufs = bufsems[:NBUF]
    gsem = bufsems[NBUF:2 * NBUF]
    ssem = bufsems[2 * NBUF:]
    wid = lax.axis_index("s") * NC + lax.axis_index("c")
    base = wid * b_per_w
    pltpu.sync_copy(idx_hbm.at[pl.ds(base, b_per_w)], idx_v)

    waiting = [None] * NBUF

    def start_gather(c, b):
      pltpu.async_copy(
          table_hbm.at[idx_v.at[pl.ds(offs[c], sizes[c])]],
          bufs[b].at[pl.ds(0, sizes[c])], gsem[b])

    def wait_gather(c, b):
      pltpu.make_async_copy(
          table_hbm.at[idx_v.at[pl.ds(0, sizes[c])]],
          bufs[b].at[pl.ds(0, sizes[c])], gsem[b]).wait()

    def start_scatter(c, b):
      waiting[b] = sizes[c]
      pltpu.async_copy(
          bufs[b].at[pl.ds(0, sizes[c])],
          out_hbm.at[pl.ds(base + offs[c], sizes[c])], ssem[b])

    def wait_scatter(b):
      n = waiting[b]
      pltpu.make_async_copy(
          bufs[b].at[pl.ds(0, n)],
          out_hbm.at[pl.ds(base, n)], ssem[b]).wait()

    for c in range(LOOKAHEAD):
      start_gather(c, c % NBUF)
    for c in range(n_chunks):
      cg = c + LOOKAHEAD
      if cg < n_chunks:
        bg = cg % NBUF
        if cg >= NBUF:
          wait_scatter(bg)
        start_gather(cg, bg)
      b = c % NBUF
      wait_gather(c, b)
      start_scatter(c, b)

    for b in range(NBUF):
      wait_scatter(b)

  return k


def kernel(tok_idx, embeddings):
  bsz, seqlen = tok_idx.shape
  flat_idx = tok_idx.reshape(bsz * seqlen)
  out = _make_gather(bsz * seqlen, DIM)(embeddings, flat_idx)
  return out.reshape(bsz, seqlen, DIM)
